# baseline (speedup 1.0000x reference)
import functools

import jax
import jax.numpy as jnp
import numpy as np
from jax.experimental import pallas as pl
from jax.experimental.pallas import tpu as pltpu

LATENT = 256
NODE = 256
TIME = 128
TSTEPS = 1000
NSPEC = 119
NSPEC_PAD = 128
CPB = 2


def _cos_schedule(T, s=0.008):
    steps = np.arange(T + 1, dtype=np.float64)
    f = np.cos(((steps / T) + s) / (1.0 + s) * np.pi / 2.0) ** 2
    ac = f / f[0]
    betas = np.clip(1.0 - ac[1:] / ac[:-1], 0.0, 0.999)
    acp = np.cumprod(1.0 - betas)
    return (np.sqrt(acp).astype(np.float32), np.sqrt(1.0 - acp).astype(np.float32))


_SQRT_AC_NP, _SQRT_1M_AC_NP = _cos_schedule(TSTEPS)


def _silu(x):
    y = 0.5 * x
    return y * jnp.tanh(y) + y


def _silu_of_2y(y):
    return y * jnp.tanh(y) + y


def _decoder_kernel(
    t_ref, na_ref, sac_ref, s1m_ref,
    z_ref, cart_ref, noise_ref, spec_ref,
    wz_ref, wc_ref, bin_ref,
    wt1_ref, bt1_ref, wt2_ref, bt2_ref, wt2n_ref,
    wa1h_ref, wa1p_ref, ba1_ref, wb1_ref, bb1_ref,
    wa2h_ref, wa2p_ref, ba2_ref, wb2_ref, bb2_ref,
    wout_ref, bout_ref, ws1_ref, bs1_ref, ws2_ref, bs2_ref,
    od_ref, os_ref, or_ref,
    *, npc: int,
):
    i = pl.program_id(0)
    f32 = jnp.float32
    bf16 = jnp.bfloat16
    rows = CPB * npc
    dot = functools.partial(jnp.dot, preferred_element_type=f32)

    rid = jax.lax.broadcasted_iota(jnp.int32, (rows, CPB), 0)
    cid = jax.lax.broadcasted_iota(jnp.int32, (rows, CPB), 1)
    S = (rid // npc == cid).astype(f32)

    sa_list = []
    s1m_list = []
    t_list = []
    for k in range(CPB):
        t_k = t_ref[i * CPB + k]
        t_list.append(t_k)
        sa_list.append(sac_ref[t_k])
        s1m_list.append(s1m_ref[t_k])

    col = jax.lax.broadcasted_iota(jnp.int32, (CPB, TIME), 1)
    half = TIME // 2
    freqs = jnp.exp(-jnp.log(f32(10000.0)) * (col % half).astype(f32) / half)
    tcol = jnp.zeros((CPB, TIME), f32)
    for k in range(CPB):
        kr = (jax.lax.broadcasted_iota(jnp.int32, (CPB, TIME), 0) == k)
        tcol = jnp.where(kr, t_list[k].astype(f32), tcol)
    arg = tcol * freqs
    temb = jnp.where(col < half, jnp.sin(arg), jnp.cos(arg))
    temb = dot(_silu(dot(temb, wt1_ref[...]) + bt1_ref[...]), wt2_ref[...]) + bt2_ref[...]
    t2n = dot(temb, wt2n_ref[...])

    sa_k = jnp.zeros((CPB, 1), f32)
    s1m_k = jnp.zeros((CPB, 1), f32)
    for k in range(CPB):
        kr = (jax.lax.broadcasted_iota(jnp.int32, (CPB, 1), 0) == k)
        sa_k = jnp.where(kr, sa_list[k], sa_k)
        s1m_k = jnp.where(kr, s1m_list[k], s1m_k)
    sa_col = dot(S, sa_k)
    s1m_col = dot(S, s1m_k)

    cart_t = sa_col * cart_ref[...] + s1m_col * noise_ref[...]
    y1 = (dot(z_ref[...].astype(bf16), wz_ref[...])
          + dot(cart_t, wc_ref[...]) + bin_ref[...] + dot(S, 0.5 * t2n))
    h = _silu_of_2y(y1)

    inv_npc = f32(1.0) / f32(npc)
    for wah, wap, ba, wb, bb in (
        (wa1h_ref, wa1p_ref, ba1_ref, wb1_ref, bb1_ref),
        (wa2h_ref, wa2p_ref, ba2_ref, wb2_ref, bb2_ref),
    ):
        pooled = jax.lax.dot_general(
            S, h, (((0,), (0,)), ((), ())), preferred_element_type=f32
        ) * inv_npc
        pba = dot(pooled, wap[...]) + ba[...]
        y = dot(h.astype(bf16), wah[...]) + dot(S, pba)
        s = _silu_of_2y(y)
        h = h + dot(s.astype(bf16), wb[...]) + bb[...]

    pred = dot(h, wout_ref[...]) + bout_ref[...]
    od_part = jnp.sum((pred - noise_ref[...]) ** 2, axis=(0, 1), keepdims=True)

    rcp_sa = dot(S, f32(1.0) / sa_k)
    f0 = (cart_t - s1m_col * pred) * rcp_sa
    cidx = jax.lax.broadcasted_iota(jnp.int32, (1, 3), 1)
    or_part = jnp.zeros((1, 1), f32)
    rr = jax.lax.broadcasted_iota(jnp.int32, (npc, npc), 0)
    cc = jax.lax.broadcasted_iota(jnp.int32, (npc, npc), 1)
    for k in range(CPB):
        fk = f0[k * npc:(k + 1) * npc]
        n_k = na_ref[i * CPB + k]
        n_f = n_k.astype(f32)
        dist_sq = None
        for j in range(3):
            ej = (cidx == j).astype(f32)
            rowj = jax.lax.dot_general(
                ej, fk, (((1,), (1,)), ((), ())), preferred_element_type=f32
            )
            d = fk[:, j:j + 1] - rowj
            dist_sq = d * d + (f32(1e-8) if dist_sq is None else dist_sq)
        dist_sq = jnp.where(rr == cc, f32(1e9), dist_sq)
        dist = jnp.sqrt(dist_sq)
        rel = jnp.maximum(f32(0.6) - dist, f32(0.0))
        rsum = jnp.sum(rel * rel, axis=(0, 1), keepdims=True)
        or_part = or_part + jnp.where(
            n_k > 1, rsum / n_f, jnp.zeros((1, 1), f32)
        )

    s1v = _silu_of_2y(dot(h, ws1_ref[...]) + bs1_ref[...])
    logits = dot(s1v, ws2_ref[...]) + bs2_ref[...]
    m = jnp.max(logits, axis=1, keepdims=True)
    lse = m + jnp.log(jnp.sum(jnp.exp(logits - m), axis=1, keepdims=True))
    lcol = jax.lax.broadcasted_iota(jnp.int32, (rows, NSPEC), 1)
    picked = jnp.sum(
        jnp.where(lcol == spec_ref[...], logits, f32(0.0)), axis=1, keepdims=True
    )
    os_part = jnp.sum(lse - picked, axis=(0, 1), keepdims=True)

    @pl.when(i == 0)
    def _():
        od_ref[...] = od_part
        os_ref[...] = os_part
        or_ref[...] = or_part

    @pl.when(i > 0)
    def _():
        od_ref[...] += od_part
        os_ref[...] += os_part
        or_ref[...] += or_part


def kernel(z_nodes, cart_coords, noise_cart, params, batch_indices, num_atoms_list, species, t):
    B = int(num_atoms_list.shape[0])
    N = int(z_nodes.shape[0])
    npc = N // B
    nblocks = B // CPB
    rows = CPB * npc
    f32 = jnp.float32
    bf16 = jnp.bfloat16

    w_in = params['W_in']
    wz = (0.5 * w_in[:LATENT]).astype(bf16)
    wc = 0.5 * w_in[LATENT:]
    bin_half = 0.5 * params['b_in']
    wa1 = params['Wb1a']
    wa2 = params['Wb2a']

    row = lambda v: v.reshape(1, -1)
    sac = jnp.asarray(_SQRT_AC_NP)
    s1m = jnp.asarray(_SQRT_1M_AC_NP)

    smem = pl.BlockSpec(memory_space=pltpu.SMEM)
    const = lambda shape: pl.BlockSpec(shape, lambda i: (0,) * len(shape))

    grid_spec = pl.GridSpec(
        grid=(nblocks,),
        in_specs=[
            smem, smem, smem, smem,
            pl.BlockSpec((rows, LATENT), lambda i: (i, 0)),
            pl.BlockSpec((rows, 3), lambda i: (i, 0)),
            pl.BlockSpec((rows, 3), lambda i: (i, 0)),
            pl.BlockSpec((rows, 1), lambda i: (i, 0)),
            const((LATENT, NODE)), const((3, NODE)), const((1, NODE)),
            const((TIME, 4 * TIME)), const((1, 4 * TIME)),
            const((4 * TIME, TIME)), const((1, TIME)), const((TIME, NODE)),
            const((NODE, 4 * NODE)), const((NODE, 4 * NODE)), const((1, 4 * NODE)),
            const((4 * NODE, NODE)), const((1, NODE)),
            const((NODE, 4 * NODE)), const((NODE, 4 * NODE)), const((1, 4 * NODE)),
            const((4 * NODE, NODE)), const((1, NODE)),
            const((NODE, 3)), const((1, 3)),
            const((NODE, NSPEC_PAD)), const((1, NSPEC_PAD)),
            const((NSPEC_PAD, NSPEC)), const((1, NSPEC)),
        ],
        out_specs=[
            pl.BlockSpec((1, 1), lambda i: (0, 0)),
            pl.BlockSpec((1, 1), lambda i: (0, 0)),
            pl.BlockSpec((1, 1), lambda i: (0, 0)),
        ],
    )

    od, os_, orr = pl.pallas_call(
        functools.partial(_decoder_kernel, npc=npc),
        grid_spec=grid_spec,
        out_shape=[jax.ShapeDtypeStruct((1, 1), f32)] * 3,
    )(
        t.astype(jnp.int32), num_atoms_list.astype(jnp.int32), sac, s1m,
        z_nodes, cart_coords, noise_cart,
        species.astype(jnp.int32).reshape(N, 1),
        wz, wc, row(bin_half),
        params['Wt1'], row(params['bt1']), params['Wt2'], row(params['bt2']),
        params['W_t2n'],
        (0.5 * wa1[:NODE]).astype(bf16), 0.5 * wa1[NODE:], row(0.5 * params['bb1a']),
        params['Wb1b'].astype(bf16), row(params['bb1b']),
        (0.5 * wa2[:NODE]).astype(bf16), 0.5 * wa2[NODE:], row(0.5 * params['bb2a']),
        params['Wb2b'].astype(bf16), row(params['bb2b']),
        params['W_out'], row(params['b_out']),
        0.5 * params['Ws1'], row(0.5 * params['bs1']), params['Ws2'], row(params['bs2']),
    )

    loss_diff = od[0, 0] / f32(N * 3)
    loss_species = os_[0, 0] / f32(N)
    l_rep = orr[0, 0] / f32(B)
    return loss_diff, loss_species, l_rep
